# baseline (speedup 1.0000x reference)
import jax
import jax.numpy as jnp
import numpy as np
from jax.experimental import pallas as pl
from jax.experimental.pallas import tpu as pltpu

N = 4096
B = 8
K = 5
G = 10
C = 10
W = B * G
CBP = 1024
NB = N // CBP
PAIRS = [(i, j) for i in range(NB) for j in range(i, NB)]
NPAIR = len(PAIRS)
NSTEP = 11
_I_NP = np.array([p[0] for p in PAIRS], dtype=np.int32)
_J_NP = np.array([p[1] for p in PAIRS], dtype=np.int32)


def _blk(v):
    return pl.ds(v * CBP, CBP)


def _cast_t1_body(ii_ref, jj_ref, L_ref, x0_ref, tri_ref, y1_ref, yacc):
    t = pl.program_id(0)
    iv = ii_ref[t]
    jv = jj_ref[t]
    Lb = L_ref[...].astype(jnp.bfloat16)
    tri_ref[0] = Lb

    @pl.when(t == 0)
    def _():
        yacc[...] = jnp.zeros((W, N), jnp.float32)

    xi = x0_ref[:, _blk(iv)].astype(jnp.bfloat16)
    yacc[:, _blk(jv)] += jnp.dot(xi, Lb, preferred_element_type=jnp.float32)

    @pl.when(iv != jv)
    def _():
        xj = x0_ref[:, _blk(jv)].astype(jnp.bfloat16)
        yacc[:, _blk(iv)] += jax.lax.dot_general(
            xj, Lb, (((1,), (1,)), ((), ())),
            preferred_element_type=jnp.float32)

    @pl.when(t == NPAIR - 1)
    def _():
        y1_ref[...] = yacc[...]


def _gcn_body(ii_ref, jj_ref, tri_ref, x0_ref, y1_ref, wseq_ref, bseq_ref,
              h_ref, P0, P1, Tbf, acc, yacc):
    s = pl.program_id(0)
    t = pl.program_id(1)
    p = jax.lax.rem(s + 1, 4)
    k = p + 1
    iv = ii_ref[t]
    jv = jj_ref[t]

    @pl.when(jnp.logical_and(s == 0, t == 0))
    def _():
        t0 = x0_ref[...]
        t1 = y1_ref[...]
        P0[...] = t0
        P1[...] = t1
        Tbf[...] = t1.astype(jnp.bfloat16)
        acc[...] = jnp.dot(wseq_ref[0, 1], t0,
                           preferred_element_type=jnp.float32) \
            + jnp.dot(wseq_ref[0, 2], t1,
                      preferred_element_type=jnp.float32)
        yacc[...] = jnp.zeros((W, N), jnp.float32)

    Lb = tri_ref[0]
    ti = Tbf[:, _blk(iv)]
    yacc[:, _blk(jv)] += jnp.dot(ti, Lb, preferred_element_type=jnp.float32)

    @pl.when(iv != jv)
    def _():
        tj = Tbf[:, _blk(jv)]
        yacc[:, _blk(iv)] += jax.lax.dot_general(
            tj, Lb, (((1,), (1,)), ((), ())),
            preferred_element_type=jnp.float32)

    @pl.when(t == NPAIR - 1)
    def _():
        @pl.when(jax.lax.rem(k, 2) == 0)
        def _():
            P0[...] = 2.0 * yacc[...] - P0[...]

        @pl.when(jax.lax.rem(k, 2) == 1)
        def _():
            P1[...] = jnp.where(k == 1, yacc[...], 2.0 * yacc[...] - P1[...])

        tk = jnp.where((jax.lax.rem(k, 2) == 0)[None, None], P0[...], P1[...])
        a = acc[...] + jnp.dot(wseq_ref[0, 0], tk,
                               preferred_element_type=jnp.float32)

        @pl.when(jnp.logical_and(k == 4, s < NSTEP - 1))
        def _():
            t0 = jax.nn.relu(a + bseq_ref[0])
            P0[...] = t0
            Tbf[...] = t0.astype(jnp.bfloat16)
            acc[...] = jnp.dot(wseq_ref[0, 1], t0,
                               preferred_element_type=jnp.float32)

        @pl.when(k < 4)
        def _():
            acc[...] = a
            Tbf[...] = tk.astype(jnp.bfloat16)

        @pl.when(s == NSTEP - 1)
        def _():
            h_ref[...] = jax.nn.relu(a + bseq_ref[0])

        yacc[...] = jnp.zeros((W, N), jnp.float32)


def _fc_body(h_ref, fcw_ref, fcb_ref, out_ref):
    logits = jnp.dot(h_ref[...], fcw_ref[...],
                     preferred_element_type=jnp.float32) + fcb_ref[...]
    m = jnp.max(logits, axis=1, keepdims=True)
    s = jnp.log(jnp.sum(jnp.exp(logits - m), axis=1, keepdims=True))
    out_ref[...] = logits - (m + s)


@jax.jit
def kernel(x, L, W1, b1, W2, b2, W3, b3, fc_w, fc_b):
    f32 = jnp.float32
    eyeB = jnp.eye(B, dtype=f32)
    w1_bd = jnp.einsum('ab,kfg->kafbg', eyeB, W1).reshape(K, B, W)
    w2_bd = jnp.einsum('ab,kfg->kafbg', eyeB, W2).reshape(K, W, W)
    w3_bd = jnp.einsum('ab,kfg->kafbg', eyeB, W3).reshape(K, W, W)
    w1t = jnp.pad(jnp.transpose(w1_bd, (0, 2, 1)), ((0, 0), (0, 0), (0, W - B)))
    w2t = jnp.transpose(w2_bd, (0, 2, 1))
    w3t = jnp.transpose(w3_bd, (0, 2, 1))
    wt = [w1t, w2t, w3t]
    biases = [jnp.tile(b1, B)[:, None], jnp.tile(b2, B)[:, None],
              jnp.tile(b3, B)[:, None]]

    zw = jnp.zeros((W, W), f32)
    zb = jnp.zeros((W, 1), f32)
    wseq, bseq = [], []
    for s in range(NSTEP):
        l, k = (s + 1) // 4, (s + 1) % 4 + 1
        slot0 = wt[l][k]
        slot1 = wt[l + 1][0] if (k == 4 and s < NSTEP - 1) else zw
        slot2 = zw
        if s == 0:
            slot1, slot2 = wt[0][0], wt[0][1]
        wseq.append(jnp.stack([slot0, slot1, slot2]))
        bseq.append(biases[l] if k == 4 else zb)
    wseq = jnp.stack(wseq)
    bseq = jnp.stack(bseq)

    x0 = jnp.pad(x[:, :, 0], ((0, W - B), (0, 0)))
    ii = jnp.asarray(_I_NP)
    jj = jnp.asarray(_J_NP)

    tri, y1 = pl.pallas_call(
        _cast_t1_body,
        grid_spec=pltpu.PrefetchScalarGridSpec(
            num_scalar_prefetch=2,
            grid=(NPAIR,),
            in_specs=[
                pl.BlockSpec((CBP, CBP), lambda t, I, J: (I[t], J[t])),
                pl.BlockSpec((W, N), lambda t, I, J: (0, 0)),
            ],
            out_specs=[
                pl.BlockSpec((1, CBP, CBP), lambda t, I, J: (t, 0, 0)),
                pl.BlockSpec((W, N), lambda t, I, J: (0, 0)),
            ],
            scratch_shapes=[pltpu.VMEM((W, N), f32)],
        ),
        out_shape=[jax.ShapeDtypeStruct((NPAIR, CBP, CBP), jnp.bfloat16),
                   jax.ShapeDtypeStruct((W, N), f32)],
    )(ii, jj, L, x0)

    h3t = pl.pallas_call(
        _gcn_body,
        grid_spec=pltpu.PrefetchScalarGridSpec(
            num_scalar_prefetch=2,
            grid=(NSTEP, NPAIR),
            in_specs=[
                pl.BlockSpec((1, CBP, CBP), lambda s, t, I, J: (t, 0, 0)),
                pl.BlockSpec((W, N), lambda s, t, I, J: (0, 0)),
                pl.BlockSpec((W, N), lambda s, t, I, J: (0, 0)),
                pl.BlockSpec((1, 3, W, W), lambda s, t, I, J: (s, 0, 0, 0)),
                pl.BlockSpec((1, W, 1), lambda s, t, I, J: (s, 0, 0)),
            ],
            out_specs=pl.BlockSpec((W, N), lambda s, t, I, J: (0, 0)),
            scratch_shapes=[
                pltpu.VMEM((W, N), f32),
                pltpu.VMEM((W, N), f32),
                pltpu.VMEM((W, N), jnp.bfloat16),
                pltpu.VMEM((W, N), f32),
                pltpu.VMEM((W, N), f32),
            ],
        ),
        out_shape=jax.ShapeDtypeStruct((W, N), f32),
    )(ii, jj, tri, x0, y1, wseq, bseq)

    ht = h3t.reshape(B, G, N).transpose(0, 2, 1).reshape(B, N * G)
    return pl.pallas_call(
        _fc_body,
        out_shape=jax.ShapeDtypeStruct((B, C), jnp.float32),
    )(ht, fc_w, fc_b[None, :])


# speedup vs baseline: 1.1868x; 1.1868x over previous
import jax
import jax.numpy as jnp
from jax.experimental import pallas as pl
from jax.experimental.pallas import tpu as pltpu

N = 4096
B = 8
K = 5
G = 10
C = 10
W = B * G
CB = 2048
CCB = 1024
NSTEP = 11
NBLK = N // CB


def _cast_t1_body(L_ref, x0_ref, Lbf_ref, y1_ref):
    Lb = L_ref[...].astype(jnp.bfloat16)
    Lbf_ref[...] = Lb
    y1_ref[...] = jnp.dot(x0_ref[...].astype(jnp.bfloat16), Lb,
                          preferred_element_type=jnp.float32)


def _gcn_body(L_ref, x0_ref, y1_ref, wseq_ref, bseq_ref, wlast_ref,
              blast_ref, h_ref, P0, P1, Tbf, acc):
    s = pl.program_id(0)
    j = pl.program_id(1)
    q = jax.lax.rem(s + 1, 4)

    @pl.when(jnp.logical_and(s == 0, j == 0))
    def _():
        t0 = x0_ref[...]
        t1 = y1_ref[...]
        P0[...] = t0
        P1[...] = t1
        Tbf[...] = t1.astype(jnp.bfloat16)
        acc[...] = jnp.dot(wseq_ref[0, 1], t0,
                           preferred_element_type=jnp.float32) \
            + jnp.dot(wseq_ref[0, 0], t1,
                      preferred_element_type=jnp.float32)

    @pl.when(jnp.logical_and(q == 0, j == 0))
    def _():
        a = acc[...] + jnp.dot(wseq_ref[0, 0], P0[...],
                               preferred_element_type=jnp.float32)
        t0 = jax.nn.relu(a + bseq_ref[0])
        P0[...] = t0
        Tbf[...] = t0.astype(jnp.bfloat16)
        acc[...] = jnp.dot(wseq_ref[0, 1], t0,
                           preferred_element_type=jnp.float32)

    @pl.when(jnp.logical_and(jnp.logical_and(jax.lax.rem(q, 2) == 1, s > 0),
                             j == 0))
    def _():
        acc[...] += jnp.dot(wseq_ref[0, 0], P1[...],
                            preferred_element_type=jnp.float32)
        Tbf[...] = P1[...].astype(jnp.bfloat16)

    @pl.when(jnp.logical_and(q == 2, j == 0))
    def _():
        acc[...] += jnp.dot(wseq_ref[0, 0], P0[...],
                            preferred_element_type=jnp.float32)
        Tbf[...] = P0[...].astype(jnp.bfloat16)

    D = jnp.dot(Tbf[...], L_ref[...], preferred_element_type=jnp.float32)
    blk = pl.ds(j * CB, CB)

    @pl.when(jax.lax.rem(q, 2) == 0)
    def _():
        P1[:, blk] = jnp.where(q == 0, D, 2.0 * D - P1[:, blk])

    @pl.when(jax.lax.rem(q, 2) == 1)
    def _():
        P0[:, blk] = 2.0 * D - P0[:, blk]

    @pl.when(s == NSTEP - 1)
    def _():
        t4b = P0[:, blk]
        h_ref[:, blk] = jax.nn.relu(
            acc[:, blk] + jnp.dot(wlast_ref[...], t4b,
                                  preferred_element_type=jnp.float32)
            + blast_ref[...])


def _fc_body(h_ref, fcw_ref, fcb_ref, out_ref):
    logits = jnp.dot(h_ref[...], fcw_ref[...],
                     preferred_element_type=jnp.float32) + fcb_ref[...]
    m = jnp.max(logits, axis=1, keepdims=True)
    s = jnp.log(jnp.sum(jnp.exp(logits - m), axis=1, keepdims=True))
    out_ref[...] = logits - (m + s)


@jax.jit
def kernel(x, L, W1, b1, W2, b2, W3, b3, fc_w, fc_b):
    f32 = jnp.float32
    eyeB = jnp.eye(B, dtype=f32)
    w1_bd = jnp.einsum('ab,kfg->kafbg', eyeB, W1).reshape(K, B, W)
    w2_bd = jnp.einsum('ab,kfg->kafbg', eyeB, W2).reshape(K, W, W)
    w3_bd = jnp.einsum('ab,kfg->kafbg', eyeB, W3).reshape(K, W, W)
    w1t = jnp.pad(jnp.transpose(w1_bd, (0, 2, 1)), ((0, 0), (0, 0), (0, W - B)))
    w2t = jnp.transpose(w2_bd, (0, 2, 1))
    w3t = jnp.transpose(w3_bd, (0, 2, 1))
    wt = [w1t, w2t, w3t]
    biases = [jnp.tile(b1, B)[:, None], jnp.tile(b2, B)[:, None],
              jnp.tile(b3, B)[:, None]]

    zw = jnp.zeros((W, W), f32)
    wseq, bseq = [], []
    for s in range(NSTEP):
        q, l = (s + 1) % 4, (s + 1) // 4
        if s == 0:
            wseq.append(jnp.stack([wt[0][1], wt[0][0]]))
            bseq.append(jnp.zeros((W, 1), f32))
        elif q == 0:
            wseq.append(jnp.stack([wt[l - 1][4], wt[l][0]]))
            bseq.append(biases[l - 1])
        else:
            wseq.append(jnp.stack([wt[l][q], zw]))
            bseq.append(jnp.zeros((W, 1), f32))
    wseq = jnp.stack(wseq)
    bseq = jnp.stack(bseq)

    x0 = jnp.pad(x[:, :, 0], ((0, W - B), (0, 0)))

    Lbf, y1 = pl.pallas_call(
        _cast_t1_body,
        grid=(N // CCB,),
        in_specs=[pl.BlockSpec((N, CCB), lambda j: (0, j)),
                  pl.BlockSpec((W, N), lambda j: (0, 0))],
        out_specs=[pl.BlockSpec((N, CCB), lambda j: (0, j)),
                   pl.BlockSpec((W, CCB), lambda j: (0, j))],
        out_shape=[jax.ShapeDtypeStruct((N, N), jnp.bfloat16),
                   jax.ShapeDtypeStruct((W, N), f32)],
    )(L, x0)

    h3t = pl.pallas_call(
        _gcn_body,
        grid=(NSTEP, NBLK),
        in_specs=[
            pl.BlockSpec((N, CB), lambda s, j: (0, j)),
            pl.BlockSpec((W, N), lambda s, j: (0, 0)),
            pl.BlockSpec((W, N), lambda s, j: (0, 0)),
            pl.BlockSpec((1, 2, W, W), lambda s, j: (s, 0, 0, 0)),
            pl.BlockSpec((1, W, 1), lambda s, j: (s, 0, 0)),
            pl.BlockSpec((W, W), lambda s, j: (0, 0)),
            pl.BlockSpec((W, 1), lambda s, j: (0, 0)),
        ],
        out_specs=pl.BlockSpec((W, N), lambda s, j: (0, 0)),
        out_shape=jax.ShapeDtypeStruct((W, N), f32),
        scratch_shapes=[
            pltpu.VMEM((W, N), f32),
            pltpu.VMEM((W, N), f32),
            pltpu.VMEM((W, N), jnp.bfloat16),
            pltpu.VMEM((W, N), f32),
        ],
    )(Lbf, x0, y1, wseq, bseq, wt[2][4], biases[2])

    ht = h3t.reshape(B, G, N).transpose(0, 2, 1).reshape(B, N * G)
    return pl.pallas_call(
        _fc_body,
        out_shape=jax.ShapeDtypeStruct((B, C), jnp.float32),
    )(ht, fc_w, fc_b[None, :])


# speedup vs baseline: 1.1877x; 1.0008x over previous
import jax
import jax.numpy as jnp
from jax.experimental import pallas as pl
from jax.experimental.pallas import tpu as pltpu

N = 4096
B = 8
K = 5
G = 10
C = 10
W = B * G
CB = 2048
CCB = 512
NSTEP = 11
NBLK = N // CB


def _cast_t1_body(L_ref, x0_ref, Lbf_ref, y1_ref):
    Lb = L_ref[...].astype(jnp.bfloat16)
    Lbf_ref[...] = Lb
    y1_ref[...] = jnp.dot(x0_ref[...].astype(jnp.bfloat16), Lb,
                          preferred_element_type=jnp.float32)


def _gcn_body(L_ref, x0_ref, y1_ref, wseq_ref, bseq_ref, wlast_ref,
              blast_ref, h_ref, P0, P1, Tbf, acc):
    s = pl.program_id(0)
    j = pl.program_id(1)
    q = jax.lax.rem(s + 1, 4)

    @pl.when(jnp.logical_and(s == 0, j == 0))
    def _():
        t0 = x0_ref[...]
        t1 = y1_ref[...]
        P0[...] = t0
        P1[...] = t1
        Tbf[...] = t1.astype(jnp.bfloat16)
        acc[...] = jnp.dot(wseq_ref[0, 1], t0,
                           preferred_element_type=jnp.float32) \
            + jnp.dot(wseq_ref[0, 0], t1,
                      preferred_element_type=jnp.float32)

    @pl.when(jnp.logical_and(q == 0, j == 0))
    def _():
        a = acc[...] + jnp.dot(wseq_ref[0, 0], P0[...],
                               preferred_element_type=jnp.float32)
        t0 = jax.nn.relu(a + bseq_ref[0])
        P0[...] = t0
        Tbf[...] = t0.astype(jnp.bfloat16)
        acc[...] = jnp.dot(wseq_ref[0, 1], t0,
                           preferred_element_type=jnp.float32)

    @pl.when(jnp.logical_and(jnp.logical_and(jax.lax.rem(q, 2) == 1, s > 0),
                             j == 0))
    def _():
        acc[...] += jnp.dot(wseq_ref[0, 0], P1[...],
                            preferred_element_type=jnp.float32)
        Tbf[...] = P1[...].astype(jnp.bfloat16)

    @pl.when(jnp.logical_and(q == 2, j == 0))
    def _():
        acc[...] += jnp.dot(wseq_ref[0, 0], P0[...],
                            preferred_element_type=jnp.float32)
        Tbf[...] = P0[...].astype(jnp.bfloat16)

    D = jnp.dot(Tbf[...], L_ref[...], preferred_element_type=jnp.float32)
    blk = pl.ds(j * CB, CB)

    @pl.when(jax.lax.rem(q, 2) == 0)
    def _():
        P1[:, blk] = jnp.where(q == 0, D, 2.0 * D - P1[:, blk])

    @pl.when(jax.lax.rem(q, 2) == 1)
    def _():
        P0[:, blk] = 2.0 * D - P0[:, blk]

    @pl.when(s == NSTEP - 1)
    def _():
        t4b = P0[:, blk]
        h_ref[:, blk] = jax.nn.relu(
            acc[:, blk] + jnp.dot(wlast_ref[...], t4b,
                                  preferred_element_type=jnp.float32)
            + blast_ref[...])


def _fc_body(h_ref, fcw_ref, fcb_ref, out_ref):
    logits = jnp.dot(h_ref[...], fcw_ref[...],
                     preferred_element_type=jnp.float32) + fcb_ref[...]
    m = jnp.max(logits, axis=1, keepdims=True)
    s = jnp.log(jnp.sum(jnp.exp(logits - m), axis=1, keepdims=True))
    out_ref[...] = logits - (m + s)


@jax.jit
def kernel(x, L, W1, b1, W2, b2, W3, b3, fc_w, fc_b):
    f32 = jnp.float32
    eyeB = jnp.eye(B, dtype=f32)
    w1_bd = jnp.einsum('ab,kfg->kafbg', eyeB, W1).reshape(K, B, W)
    w2_bd = jnp.einsum('ab,kfg->kafbg', eyeB, W2).reshape(K, W, W)
    w3_bd = jnp.einsum('ab,kfg->kafbg', eyeB, W3).reshape(K, W, W)
    w1t = jnp.pad(jnp.transpose(w1_bd, (0, 2, 1)), ((0, 0), (0, 0), (0, W - B)))
    w2t = jnp.transpose(w2_bd, (0, 2, 1))
    w3t = jnp.transpose(w3_bd, (0, 2, 1))
    wt = [w1t, w2t, w3t]
    biases = [jnp.tile(b1, B)[:, None], jnp.tile(b2, B)[:, None],
              jnp.tile(b3, B)[:, None]]

    zw = jnp.zeros((W, W), f32)
    wseq, bseq = [], []
    for s in range(NSTEP):
        q, l = (s + 1) % 4, (s + 1) // 4
        if s == 0:
            wseq.append(jnp.stack([wt[0][1], wt[0][0]]))
            bseq.append(jnp.zeros((W, 1), f32))
        elif q == 0:
            wseq.append(jnp.stack([wt[l - 1][4], wt[l][0]]))
            bseq.append(biases[l - 1])
        else:
            wseq.append(jnp.stack([wt[l][q], zw]))
            bseq.append(jnp.zeros((W, 1), f32))
    wseq = jnp.stack(wseq)
    bseq = jnp.stack(bseq)

    x0 = jnp.pad(x[:, :, 0], ((0, W - B), (0, 0)))

    Lbf, y1 = pl.pallas_call(
        _cast_t1_body,
        grid=(N // CCB,),
        in_specs=[pl.BlockSpec((N, CCB), lambda j: (0, j)),
                  pl.BlockSpec((W, N), lambda j: (0, 0))],
        out_specs=[pl.BlockSpec((N, CCB), lambda j: (0, j)),
                   pl.BlockSpec((W, CCB), lambda j: (0, j))],
        out_shape=[jax.ShapeDtypeStruct((N, N), jnp.bfloat16),
                   jax.ShapeDtypeStruct((W, N), f32)],
    )(L, x0)

    h3t = pl.pallas_call(
        _gcn_body,
        grid=(NSTEP, NBLK),
        in_specs=[
            pl.BlockSpec((N, CB), lambda s, j: (0, j)),
            pl.BlockSpec((W, N), lambda s, j: (0, 0)),
            pl.BlockSpec((W, N), lambda s, j: (0, 0)),
            pl.BlockSpec((1, 2, W, W), lambda s, j: (s, 0, 0, 0)),
            pl.BlockSpec((1, W, 1), lambda s, j: (s, 0, 0)),
            pl.BlockSpec((W, W), lambda s, j: (0, 0)),
            pl.BlockSpec((W, 1), lambda s, j: (0, 0)),
        ],
        out_specs=pl.BlockSpec((W, N), lambda s, j: (0, 0)),
        out_shape=jax.ShapeDtypeStruct((W, N), f32),
        scratch_shapes=[
            pltpu.VMEM((W, N), f32),
            pltpu.VMEM((W, N), f32),
            pltpu.VMEM((W, N), jnp.bfloat16),
            pltpu.VMEM((W, N), f32),
        ],
    )(Lbf, x0, y1, wseq, bseq, wt[2][4], biases[2])

    ht = h3t.reshape(B, G, N).transpose(0, 2, 1).reshape(B, N * G)
    return pl.pallas_call(
        _fc_body,
        out_shape=jax.ShapeDtypeStruct((B, C), jnp.float32),
    )(ht, fc_w, fc_b[None, :])
